# baseline (speedup 1.0000x reference)
import functools

import jax
import jax.numpy as jnp
from jax import lax
from jax.experimental import pallas as pl
from jax.experimental.pallas import tpu as pltpu
from jax.experimental.pallas import tpu_sc as plsc


def _argmax_body(x_ref, idx_ref):
    x = x_ref[...]
    m = jnp.max(x, axis=1, keepdims=True)
    ii = lax.broadcasted_iota(jnp.int32, x.shape, 1)
    idx_ref[...] = jnp.min(jnp.where(x == m, ii, jnp.int32(2**30)), axis=1,
                           keepdims=True)


def _tc_argmax(lat2):
    R, C = lat2.shape
    BR = 4096
    return pl.pallas_call(
        _argmax_body,
        grid=(R // BR,),
        in_specs=[pl.BlockSpec((BR, C), lambda i: (i, 0))],
        out_specs=pl.BlockSpec((BR, 1), lambda i: (i, 0)),
        out_shape=jax.ShapeDtypeStruct((R, 1), jnp.int32),
    )(lat2)


def _tr_body(x_ref, o_ref):
    D = o_ref.shape[0]
    for b in range(x_ref.shape[0]):
        o_ref[:, b, :] = x_ref[b, :, :D].T


def _tr_body_acc(x_ref, prev_ref, o_ref):
    del prev_ref
    _tr_body(x_ref, o_ref)


def _tc_transpose(x3, D, B, h, prev):
    H, K, DP = x3.shape
    BK = 128
    out_shape = jax.ShapeDtypeStruct((D, B, K), jnp.float32)
    out_spec = pl.BlockSpec((D, H, BK), lambda j: (0, h, j))
    in_spec = pl.BlockSpec((H, BK, DP), lambda j: (0, j, 0))
    if prev is None:
        return pl.pallas_call(
            _tr_body,
            grid=(K // BK,),
            in_specs=[in_spec],
            out_specs=out_spec,
            out_shape=out_shape,
        )(x3)
    return pl.pallas_call(
        _tr_body_acc,
        grid=(K // BK,),
        in_specs=[in_spec,
                  pl.BlockSpec(memory_space=pltpu.MemorySpace.HBM)],
        out_specs=out_spec,
        out_shape=out_shape,
        input_output_aliases={1: 0},
    )(x3, prev)


def _sc_gather(basis_p, idx, R, D):
    info = plsc.get_sparse_core_info()
    NC, NS = info.num_cores, info.num_subcores
    NW = NC * NS
    b_per_w = R // NW
    CP = 32
    n_ch = b_per_w // CP
    DP = basis_p.shape[1]
    idx3 = idx.reshape(NW, n_ch, CP)

    mesh = plsc.VectorSubcoreMesh(core_axis_name="c", subcore_axis_name="s")

    @functools.partial(
        pl.kernel, mesh=mesh,
        out_type=jax.ShapeDtypeStruct((R, DP), jnp.float32),
        scratch_types=[
            pltpu.VMEM((n_ch, CP), jnp.int32),
            pltpu.VMEM((2, CP, DP), jnp.float32),
            pltpu.SemaphoreType.DMA,
            pltpu.SemaphoreType.DMA,
        ],
    )
    def gather_k(basis_hbm, idx_hbm, out_hbm, idx_v, rows_v, sem0, sem1):
        wid = lax.axis_index("s") * NC + lax.axis_index("c")
        base = wid * b_per_w
        sems = (sem0, sem1)
        pltpu.sync_copy(idx_hbm.at[wid], idx_v)
        cps = [None, None]
        cps[0] = pltpu.async_copy(
            basis_hbm.at[idx_v.at[0]], rows_v.at[0], sems[0])
        for c in range(n_ch):
            if c + 1 < n_ch:
                cps[(c + 1) % 2] = pltpu.async_copy(
                    basis_hbm.at[idx_v.at[c + 1]],
                    rows_v.at[(c + 1) % 2], sems[(c + 1) % 2])
            cps[c % 2].wait()
            pltpu.sync_copy(rows_v.at[c % 2],
                            out_hbm.at[pl.ds(base + c * CP, CP)])

    return gather_k(basis_p, idx3)


def kernel(latent_coeffs, basis_vectors):
    B, K, C = latent_coeffs.shape
    V, D = basis_vectors.shape
    NH = 2
    H = B // NH
    basis_p = jnp.pad(basis_vectors, ((0, 0), (0, 1024 - D)))
    q_t = None
    idxs = []
    for h in range(NH):
        lat_h = latent_coeffs[h * H:(h + 1) * H].reshape(H * K, C)
        idx_h = _tc_argmax(lat_h).reshape(H * K)
        padded_h = _sc_gather(basis_p, idx_h, H * K, D)
        q_t = _tc_transpose(padded_h.reshape(H, K, -1), D, B, h, q_t)
        idxs.append(idx_h.reshape(H, K))
    quant = jnp.transpose(q_t, (1, 2, 0))
    return (quant, jnp.concatenate(idxs, axis=0))


# speedup vs baseline: 1.2953x; 1.2953x over previous
import functools

import jax
import jax.numpy as jnp
from jax import lax
from jax.experimental import pallas as pl
from jax.experimental.pallas import tpu as pltpu
from jax.experimental.pallas import tpu_sc as plsc


def _argmax_body(x_ref, idx_ref):
    x = x_ref[...]
    m = jnp.max(x, axis=1, keepdims=True)
    ii = lax.broadcasted_iota(jnp.int32, x.shape, 1)
    idx_ref[...] = jnp.min(jnp.where(x == m, ii, jnp.int32(2**30)), axis=1,
                           keepdims=True)


def _tc_argmax(lat2, h, nh):
    R, C = lat2.shape
    RH = R // nh
    BR = 2048
    return pl.pallas_call(
        _argmax_body,
        grid=(RH // BR,),
        in_specs=[pl.BlockSpec((BR, C), lambda i: (i + h * (RH // BR), 0))],
        out_specs=pl.BlockSpec((BR, 1), lambda i: (i, 0)),
        out_shape=jax.ShapeDtypeStruct((RH, 1), jnp.int32),
    )(lat2)


def _tr_body(x_ref, o_ref):
    D = o_ref.shape[0]
    for b in range(x_ref.shape[0]):
        o_ref[:, b, :] = x_ref[b, :, :D].T


def _tr_body_acc(x_ref, prev_ref, o_ref):
    del prev_ref
    _tr_body(x_ref, o_ref)


def _tc_transpose(x3, D, B, h, prev):
    H, K, DP = x3.shape
    BK = 128
    out_shape = jax.ShapeDtypeStruct((D, B, K), jnp.float32)
    out_spec = pl.BlockSpec((D, H, BK), lambda j: (0, h, j))
    in_spec = pl.BlockSpec((H, BK, DP), lambda j: (0, j, 0))
    if prev is None:
        return pl.pallas_call(
            _tr_body,
            grid=(K // BK,),
            in_specs=[in_spec],
            out_specs=out_spec,
            out_shape=out_shape,
        )(x3)
    return pl.pallas_call(
        _tr_body_acc,
        grid=(K // BK,),
        in_specs=[in_spec,
                  pl.BlockSpec(memory_space=pltpu.MemorySpace.HBM)],
        out_specs=out_spec,
        out_shape=out_shape,
        input_output_aliases={1: 0},
    )(x3, prev)


def _sc_gather(basis_p, idx, R, D):
    info = plsc.get_sparse_core_info()
    NC, NS = info.num_cores, info.num_subcores
    NW = NC * NS
    b_per_w = R // NW
    CP = 32
    n_ch = b_per_w // CP
    DP = basis_p.shape[1]
    idx3 = idx.reshape(NW, n_ch, CP)

    mesh = plsc.VectorSubcoreMesh(core_axis_name="c", subcore_axis_name="s")

    @functools.partial(
        pl.kernel, mesh=mesh,
        out_type=jax.ShapeDtypeStruct((R, DP), jnp.float32),
        scratch_types=[
            pltpu.VMEM((n_ch, CP), jnp.int32),
            pltpu.VMEM((2, CP, DP), jnp.float32),
            pltpu.SemaphoreType.DMA,
            pltpu.SemaphoreType.DMA,
        ],
    )
    def gather_k(basis_hbm, idx_hbm, out_hbm, idx_v, rows_v, sem0, sem1):
        wid = lax.axis_index("s") * NC + lax.axis_index("c")
        base = wid * b_per_w
        sems = (sem0, sem1)
        pltpu.sync_copy(idx_hbm.at[wid], idx_v)
        cps = [None, None]
        cps[0] = pltpu.async_copy(
            basis_hbm.at[idx_v.at[0]], rows_v.at[0], sems[0])
        for c in range(n_ch):
            if c + 1 < n_ch:
                cps[(c + 1) % 2] = pltpu.async_copy(
                    basis_hbm.at[idx_v.at[c + 1]],
                    rows_v.at[(c + 1) % 2], sems[(c + 1) % 2])
            cps[c % 2].wait()
            pltpu.sync_copy(rows_v.at[c % 2],
                            out_hbm.at[pl.ds(base + c * CP, CP)])

    return gather_k(basis_p, idx3)


def kernel(latent_coeffs, basis_vectors):
    B, K, C = latent_coeffs.shape
    V, D = basis_vectors.shape
    NH = 2
    H = B // NH
    basis_p = jnp.pad(basis_vectors, ((0, 0), (0, 1024 - D)))
    q_t = None
    idxs = []
    lat2 = latent_coeffs.reshape(B * K, C)
    for h in range(NH):
        idx_h = _tc_argmax(lat2, h, NH).reshape(H * K)
        padded_h = _sc_gather(basis_p, idx_h, H * K, D)
        q_t = _tc_transpose(padded_h.reshape(H, K, -1), D, B, h, q_t)
        idxs.append(idx_h.reshape(H, K))
    quant = jnp.transpose(q_t, (1, 2, 0))
    return (quant, jnp.concatenate(idxs, axis=0))
